# Initial kernel scaffold; baseline (speedup 1.0000x reference)
import jax
import jax.numpy as jnp
from jax.experimental import pallas as pl


def kernel(node_features, edge_index, optimizer_features, hidden_state, params):
    raise NotImplementedError("write your pallas kernel here")


# speedup vs baseline: 2.2999x; 2.2999x over previous
import functools

import jax
import jax.numpy as jnp
from jax import lax
from jax.experimental import pallas as pl
from jax.experimental.pallas import tpu as pltpu
from jax.experimental.pallas import tpu_sc as plsc

N = 10000
E = 320000
NODE_IN = 128
EDGE_IN = 16
NH = 64
EH = 64
GH = 128
L = 3
R = 2
HS = 32
SCALE = 1e-3

NC, NS = 2, 16
NW = NC * NS
CH = 128
RPC = 8
CHUNK = CH * RPC
HALF = CHUNK // 2
BE = 2000

NP_ = 10240
RZ = NP_ // NS
DUMP = N + 8
EP = ((E + CHUNK - 1) // CHUNK) * CHUNK
G2E = 2 * E


@functools.lru_cache(maxsize=None)
def _mesh():
    return plsc.VectorSubcoreMesh(
        core_axis_name="c", subcore_axis_name="s",
        num_cores=NC, num_subcores=NS)


def _wid():
    return lax.axis_index("s") * NC + lax.axis_index("c")


def _build_gather(interpret=False):
    n_chunks = G2E // CHUNK
    k_max = (n_chunks + NW - 1) // NW

    @functools.partial(
        pl.kernel, mesh=_mesh(), interpret=interpret,
        out_type=jax.ShapeDtypeStruct((G2E, GH), jnp.float32),
        scratch_types=[
            pltpu.VMEM((RPC, CH), jnp.int32),
            pltpu.VMEM((HALF, GH), jnp.float32),
            pltpu.SemaphoreType.DMA,
        ],
    )
    def gather_k(table_hbm, idx_hbm, out_hbm, idx_v, rows_v, sem):
        wid = _wid()

        def body(k, carry):
            chunk = wid + k * NW

            @pl.when(chunk < n_chunks)
            def _():
                rbase = pl.multiple_of(chunk * RPC, RPC)
                pltpu.sync_copy(idx_hbm.at[pl.ds(rbase, RPC)], idx_v)
                for half in range(2):
                    descs = []
                    for j in range(RPC // 2):
                        descs.append(pltpu.async_copy(
                            table_hbm.at[idx_v.at[half * (RPC // 2) + j]],
                            rows_v.at[pl.ds(j * CH, CH)], sem))
                    for d in descs:
                        d.wait()
                    obase = pl.multiple_of(
                        chunk * CHUNK + half * HALF, HALF)
                    pltpu.sync_copy(rows_v, out_hbm.at[pl.ds(obase, HALF)])

            return carry

        lax.fori_loop(0, k_max, body, 0)

    return gather_k


def _build_segsum(const_ones, interpret=False):
    n_chunks = EP // CHUNK
    k_max = (n_chunks + NW - 1) // NW
    q_rows = 2 * CH

    @functools.partial(
        pl.kernel, mesh=_mesh(), interpret=interpret,
        out_type=jax.ShapeDtypeStruct((NC * NP_, GH), jnp.float32),
        scratch_types=[
            pltpu.VMEM((RPC, CH), jnp.int32),
            pltpu.VMEM((q_rows, GH), jnp.float32),
            pltpu.VMEM_SHARED((NP_, GH), jnp.float32),
        ],
    )
    def segsum_k(vals_hbm, idx_hbm, zeros_hbm, out_hbm, idx_v, rows_v, acc_sh):
        cid = lax.axis_index("c")
        sid = lax.axis_index("s")
        wid = _wid()
        zbase = pl.multiple_of(sid * RZ, RZ)
        pltpu.sync_copy(zeros_hbm.at[pl.ds(zbase, RZ)],
                        acc_sh.at[pl.ds(zbase, RZ)])
        if const_ones:
            pltpu.sync_copy(vals_hbm, rows_v)
        plsc.subcore_barrier()

        def body(k, carry):
            chunk = wid + k * NW

            @pl.when(chunk < n_chunks)
            def _():
                rbase = pl.multiple_of(chunk * RPC, RPC)
                pltpu.sync_copy(idx_hbm.at[pl.ds(rbase, RPC)], idx_v)
                for q in range(CHUNK // q_rows):
                    if not const_ones:
                        ebase = pl.multiple_of(
                            chunk * CHUNK + q * q_rows, q_rows)
                        pltpu.sync_copy(vals_hbm.at[pl.ds(ebase, q_rows)],
                                        rows_v)
                    for j in range(q_rows // CH):
                        pltpu.sync_copy(
                            rows_v.at[pl.ds(j * CH, CH)],
                            acc_sh.at[idx_v.at[q * (q_rows // CH) + j]],
                            add=True)

            return carry

        lax.fori_loop(0, k_max, body, 0)
        plsc.subcore_barrier()
        obase = pl.multiple_of(cid * NP_ + sid * RZ, RZ)
        pltpu.sync_copy(acc_sh.at[pl.ds(zbase, RZ)],
                        out_hbm.at[pl.ds(obase, RZ)])

    return segsum_k


def _gelu(x):
    return jax.nn.gelu(x)


def _dot(a, b):
    return jnp.dot(a, b, preferred_element_type=jnp.float32)


def _ln_apply(x, g, b):
    mu = jnp.mean(x, axis=-1, keepdims=True)
    var = jnp.mean((x - mu) ** 2, axis=-1, keepdims=True)
    return (x - mu) / jnp.sqrt(var + 1e-5) * g + b


def _proj_table(nh, wa, wb):
    return jnp.concatenate([_dot(nh, wa), _dot(nh, wb)], axis=0)


def _tc_node_proj(nf, w, b, wa, wb):
    def body(nf_r, w_r, b_r, wa_r, wb_r, o_r, p_r):
        nh = _gelu(_dot(nf_r[...], w_r[...]) + b_r[...])
        o_r[...] = nh
        p_r[...] = _proj_table(nh, wa_r[...], wb_r[...])
    return pl.pallas_call(
        body,
        out_shape=[jax.ShapeDtypeStruct((N, NH), jnp.float32),
                   jax.ShapeDtypeStruct((2 * N, GH), jnp.float32)],
    )(nf, w, b, wa, wb)


def _tc_edge_proj(opt, hid, wa, wb, b):
    def body(o_r, h_r, wa_r, wb_r, b_r, out_r):
        x = _gelu(_dot(o_r[...], wa_r[...]) +
                  _dot(h_r[...], wb_r[...]) + b_r[...])
        out_r[...] = jnp.concatenate([x, x], axis=-1)
    nb = E // BE
    return pl.pallas_call(
        body,
        grid=(nb,),
        in_specs=[
            pl.BlockSpec((BE, EDGE_IN), lambda i: (i, 0)),
            pl.BlockSpec((BE, HS), lambda i: (i, 0)),
            pl.BlockSpec((EDGE_IN, EH), lambda i: (0, 0)),
            pl.BlockSpec((HS, EH), lambda i: (0, 0)),
            pl.BlockSpec((1, EH), lambda i: (0, 0)),
        ],
        out_specs=pl.BlockSpec((BE, GH), lambda i: (i, 0)),
        out_shape=jax.ShapeDtypeStruct((EP, GH), jnp.float32),
    )(opt, hid, wa, wb, b)


def _tc_edge_mlp(gath, eh, wc, b1, w2, b2, g, bl):
    nb = E // BE

    def body(gs_r, gd_r, e_r, wc_r, b1_r, w2_r, b2_r, g_r, bl_r, out_r):
        e_v = e_r[...][:, :EH]
        h = gs_r[...] + gd_r[...] + _dot(e_v, wc_r[...]) + b1_r[...]
        m = _dot(_gelu(h), w2_r[...]) + b2_r[...]
        x = _ln_apply(e_v + m, g_r[...], bl_r[...])
        out_r[...] = jnp.concatenate([x, x], axis=-1)

    return pl.pallas_call(
        body,
        grid=(nb,),
        in_specs=[
            pl.BlockSpec((BE, GH), lambda i: (i, 0)),
            pl.BlockSpec((BE, GH), lambda i: (i + nb, 0)),
            pl.BlockSpec((BE, GH), lambda i: (i, 0)),
            pl.BlockSpec((EH, GH), lambda i: (0, 0)),
            pl.BlockSpec((1, GH), lambda i: (0, 0)),
            pl.BlockSpec((GH, EH), lambda i: (0, 0)),
            pl.BlockSpec((1, EH), lambda i: (0, 0)),
            pl.BlockSpec((1, EH), lambda i: (0, 0)),
            pl.BlockSpec((1, EH), lambda i: (0, 0)),
        ],
        out_specs=pl.BlockSpec((BE, GH), lambda i: (i, 0)),
        out_shape=jax.ShapeDtypeStruct((EP, GH), jnp.float32),
    )(gath, gath, eh, wc, b1, w2, b2, g, bl)


def _tc_deginv(degp):
    def body(d_r, o_r):
        d = d_r[...]
        o_r[...] = 1.0 / jnp.maximum(d[:N, :NH] + d[NP_:NP_ + N, :NH], 1.0)
    return pl.pallas_call(
        body, out_shape=jax.ShapeDtypeStruct((N, NH), jnp.float32),
    )(degp)


def _tc_node_mlp(nh, aggp, dinv, wd, wf, b1, w2, b2, g, bl, wa, wb):
    def body(nh_r, a_r, di_r, wd_r, wf_r, b1_r, w2_r, b2_r, g_r, bl_r,
             wa_r, wb_r, out_r, p_r):
        nh_v = nh_r[...]
        a = a_r[...]
        agg = (a[:N, :NH] + a[NP_:NP_ + N, :NH]) * di_r[...]
        h = _gelu(_dot(nh_v, wd_r[...]) + _dot(agg, wf_r[...]) + b1_r[...])
        u = _dot(h, w2_r[...]) + b2_r[...]
        nh_new = _ln_apply(nh_v + u, g_r[...], bl_r[...])
        out_r[...] = nh_new
        p_r[...] = _proj_table(nh_new, wa_r[...], wb_r[...])
    return pl.pallas_call(
        body,
        out_shape=[jax.ShapeDtypeStruct((N, NH), jnp.float32),
                   jax.ShapeDtypeStruct((2 * N, GH), jnp.float32)],
    )(nh, aggp, dinv, wd, wf, b1, w2, b2, g, bl, wa, wb)


def _tc_heads(eh, hid, wh1, bh1, wh2, bh2, wx, whh, bx, bhh):
    nb = E // BE

    def body(e_r, h_r, wh1_r, bh1_r, wh2_r, bh2_r, wx_r, whh_r, bx_r, bhh_r,
             u_r, nh_r):
        e_v = e_r[...][:, :EH]
        h_v = h_r[...]
        t = _gelu(_dot(e_v, wh1_r[...]) + bh1_r[...])
        u_r[...] = (_dot(t, wh2_r[...]) + bh2_r[...]) * SCALE
        gx = _dot(e_v, wx_r[...]) + bx_r[...]
        gh = _dot(h_v, whh_r[...]) + bhh_r[...]
        r = jax.nn.sigmoid(gx[:, :HS] + gh[:, :HS])
        z = jax.nn.sigmoid(gx[:, HS:2 * HS] + gh[:, HS:2 * HS])
        n = jnp.tanh(gx[:, 2 * HS:] + r * gh[:, 2 * HS:])
        nh_r[...] = (1.0 - z) * n + z * h_v

    return pl.pallas_call(
        body,
        grid=(nb,),
        in_specs=[
            pl.BlockSpec((BE, GH), lambda i: (i, 0)),
            pl.BlockSpec((BE, HS), lambda i: (i, 0)),
            pl.BlockSpec((EH, GH), lambda i: (0, 0)),
            pl.BlockSpec((1, GH), lambda i: (0, 0)),
            pl.BlockSpec((GH, 1), lambda i: (0, 0)),
            pl.BlockSpec((1, 1), lambda i: (0, 0)),
            pl.BlockSpec((EH, 3 * HS), lambda i: (0, 0)),
            pl.BlockSpec((HS, 3 * HS), lambda i: (0, 0)),
            pl.BlockSpec((1, 3 * HS), lambda i: (0, 0)),
            pl.BlockSpec((1, 3 * HS), lambda i: (0, 0)),
        ],
        out_specs=[
            pl.BlockSpec((BE, 1), lambda i: (i, 0)),
            pl.BlockSpec((BE, HS), lambda i: (i, 0)),
        ],
        out_shape=[
            jax.ShapeDtypeStruct((E, 1), jnp.float32),
            jax.ShapeDtypeStruct((E, HS), jnp.float32),
        ],
    )(eh, hid, wh1, bh1, wh2, bh2, wx, whh, bx, bhh)


def kernel(node_features, edge_index, optimizer_features, hidden_state, params):
    p = params
    src = edge_index[0]
    dst = edge_index[1]
    idx_all = jnp.concatenate([src, dst + N]).reshape(G2E // CH, CH)
    dst_pad = jnp.concatenate(
        [dst, jnp.full((EP - E,), DUMP, jnp.int32)]).reshape(EP // CH, CH)
    zeros_acc = jnp.zeros((NP_, GH), jnp.float32)
    ones_vals = jnp.ones((2 * CH, GH), jnp.float32)

    gather_f = _build_gather()
    segsum_f = _build_segsum(const_ones=False)
    degree_f = _build_segsum(const_ones=True)

    r1 = lambda v: v.reshape(1, -1)
    lyrs = p["layers"]
    splits = []
    for lyr in lyrs:
        splits.append((lyr["We1"][:NH], lyr["We1"][NH:2 * NH],
                       lyr["We1"][2 * NH:], lyr["Wn1"][:NH], lyr["Wn1"][NH:]))

    node_h, ptab = _tc_node_proj(node_features, p["Wnp"], r1(p["bnp"]),
                                 splits[0][0], splits[0][1])
    edge_h = _tc_edge_proj(optimizer_features, hidden_state,
                           p["Wep"][:EDGE_IN], p["Wep"][EDGE_IN:], r1(p["bep"]))

    degp = degree_f(ones_vals, dst_pad, zeros_acc)
    dinv = _tc_deginv(degp)

    n_rounds = L * R
    for t in range(n_rounds):
        lyr = lyrs[t // R]
        wa, wb, wc, wd, wf = splits[t // R]
        na, nb_ = splits[min(t + 1, n_rounds - 1) // R][:2]
        gath = gather_f(ptab, idx_all)
        edge_h = _tc_edge_mlp(gath, edge_h, wc, r1(lyr["be1"]),
                              lyr["We2"], r1(lyr["be2"]),
                              r1(lyr["ge"]), r1(lyr["be_ln"]))
        aggp = segsum_f(edge_h, dst_pad, zeros_acc)
        node_h, ptab = _tc_node_mlp(node_h, aggp, dinv, wd, wf, r1(lyr["bn1"]),
                                    lyr["Wn2"], r1(lyr["bn2"]),
                                    r1(lyr["gn"]), r1(lyr["bn_ln"]), na, nb_)

    updates, new_hidden = _tc_heads(
        edge_h, hidden_state, p["Wh1"], r1(p["bh1"]), p["Wh2"], r1(p["bh2"]),
        p["Wx"], p["Whh"], r1(p["bx"]), r1(p["bhh"]))
    return updates, new_hidden


# speedup vs baseline: 2.3831x; 1.0362x over previous
import functools

import jax
import jax.numpy as jnp
from jax import lax
from jax.experimental import pallas as pl
from jax.experimental.pallas import tpu as pltpu
from jax.experimental.pallas import tpu_sc as plsc

N = 10000
E = 320000
NODE_IN = 128
EDGE_IN = 16
NH = 64
EH = 64
GH = 128
L = 3
R = 2
HS = 32
SCALE = 1e-3

NC, NS = 2, 16
NW = NC * NS
CH = 128
RPC = 8
CHUNK = CH * RPC
HALF = CHUNK // 2
BE = 2000

NP_ = 10240
RZ = NP_ // NS
DUMP = N + 8
EP = ((E + CHUNK - 1) // CHUNK) * CHUNK
G2E = 2 * E


@functools.lru_cache(maxsize=None)
def _mesh():
    return plsc.VectorSubcoreMesh(
        core_axis_name="c", subcore_axis_name="s",
        num_cores=NC, num_subcores=NS)


def _wid():
    return lax.axis_index("s") * NC + lax.axis_index("c")


def _build_gather(interpret=False):
    n_chunks = EP // CHUNK
    k_max = (n_chunks + NW - 1) // NW

    @functools.partial(
        pl.kernel, mesh=_mesh(), interpret=interpret,
        out_type=jax.ShapeDtypeStruct((EP, GH), jnp.float32),
        scratch_types=[
            pltpu.VMEM((RPC, CH), jnp.int32),
            pltpu.VMEM((RPC, CH), jnp.int32),
            pltpu.VMEM((HALF, GH), jnp.float32),
            pltpu.SemaphoreType.DMA,
        ],
    )
    def gather_k(table_hbm, idxs_hbm, idxd_hbm, out_hbm, idxs_v, idxd_v,
                 rows_v, sem):
        wid = _wid()

        def body(k, carry):
            chunk = wid + k * NW

            @pl.when(chunk < n_chunks)
            def _():
                rbase = pl.multiple_of(chunk * RPC, RPC)
                pltpu.sync_copy(idxs_hbm.at[pl.ds(rbase, RPC)], idxs_v)
                pltpu.sync_copy(idxd_hbm.at[pl.ds(rbase, RPC)], idxd_v)
                for half in range(2):
                    descs = []
                    for j in range(RPC // 2):
                        descs.append(pltpu.async_copy(
                            table_hbm.at[idxs_v.at[half * (RPC // 2) + j]],
                            rows_v.at[pl.ds(j * CH, CH)], sem))
                    for d in descs:
                        d.wait()
                    descs = []
                    for j in range(RPC // 2):
                        descs.append(pltpu.async_copy(
                            table_hbm.at[idxd_v.at[half * (RPC // 2) + j]],
                            rows_v.at[pl.ds(j * CH, CH)], sem, add=True))
                    for d in descs:
                        d.wait()
                    obase = pl.multiple_of(
                        chunk * CHUNK + half * HALF, HALF)
                    pltpu.sync_copy(rows_v, out_hbm.at[pl.ds(obase, HALF)])

            return carry

        lax.fori_loop(0, k_max, body, 0)

    return gather_k


def _build_segsum(const_ones, interpret=False):
    n_chunks = EP // CHUNK
    k_max = (n_chunks + NW - 1) // NW
    q_rows = 2 * CH

    @functools.partial(
        pl.kernel, mesh=_mesh(), interpret=interpret,
        out_type=jax.ShapeDtypeStruct((NC * NP_, GH), jnp.float32),
        scratch_types=[
            pltpu.VMEM((RPC, CH), jnp.int32),
            pltpu.VMEM((q_rows, GH), jnp.float32),
            pltpu.VMEM_SHARED((NP_, GH), jnp.float32),
        ],
    )
    def segsum_k(vals_hbm, idx_hbm, zeros_hbm, out_hbm, idx_v, rows_v, acc_sh):
        cid = lax.axis_index("c")
        sid = lax.axis_index("s")
        wid = _wid()
        zbase = pl.multiple_of(sid * RZ, RZ)
        pltpu.sync_copy(zeros_hbm.at[pl.ds(zbase, RZ)],
                        acc_sh.at[pl.ds(zbase, RZ)])
        if const_ones:
            pltpu.sync_copy(vals_hbm, rows_v)
        plsc.subcore_barrier()

        def body(k, carry):
            chunk = wid + k * NW

            @pl.when(chunk < n_chunks)
            def _():
                rbase = pl.multiple_of(chunk * RPC, RPC)
                pltpu.sync_copy(idx_hbm.at[pl.ds(rbase, RPC)], idx_v)
                for q in range(CHUNK // q_rows):
                    if not const_ones:
                        ebase = pl.multiple_of(
                            chunk * CHUNK + q * q_rows, q_rows)
                        pltpu.sync_copy(vals_hbm.at[pl.ds(ebase, q_rows)],
                                        rows_v)
                    for j in range(q_rows // CH):
                        pltpu.sync_copy(
                            rows_v.at[pl.ds(j * CH, CH)],
                            acc_sh.at[idx_v.at[q * (q_rows // CH) + j]],
                            add=True)

            return carry

        lax.fori_loop(0, k_max, body, 0)
        plsc.subcore_barrier()
        obase = pl.multiple_of(cid * NP_ + sid * RZ, RZ)
        pltpu.sync_copy(acc_sh.at[pl.ds(zbase, RZ)],
                        out_hbm.at[pl.ds(obase, RZ)])

    return segsum_k


def _gelu(x):
    return jax.nn.gelu(x)


def _dot(a, b):
    return jnp.dot(a, b, preferred_element_type=jnp.float32)


def _ln_apply(x, g, b):
    mu = jnp.mean(x, axis=-1, keepdims=True)
    var = jnp.mean((x - mu) ** 2, axis=-1, keepdims=True)
    return (x - mu) / jnp.sqrt(var + 1e-5) * g + b


def _proj_table(nh, wa, wb):
    return jnp.concatenate([_dot(nh, wa), _dot(nh, wb)], axis=0)


def _tc_node_proj(nf, w, b, wa, wb):
    def body(nf_r, w_r, b_r, wa_r, wb_r, o_r, p_r):
        nh = _gelu(_dot(nf_r[...], w_r[...]) + b_r[...])
        o_r[...] = nh
        p_r[...] = _proj_table(nh, wa_r[...], wb_r[...])
    return pl.pallas_call(
        body,
        out_shape=[jax.ShapeDtypeStruct((N, NH), jnp.float32),
                   jax.ShapeDtypeStruct((2 * N, GH), jnp.float32)],
    )(nf, w, b, wa, wb)


def _tc_edge_proj(opt, hid, wa, wb, b):
    def body(o_r, h_r, wa_r, wb_r, b_r, out_r):
        x = _gelu(_dot(o_r[...], wa_r[...]) +
                  _dot(h_r[...], wb_r[...]) + b_r[...])
        out_r[...] = jnp.concatenate([x, x], axis=-1)
    nb = E // BE
    return pl.pallas_call(
        body,
        grid=(nb,),
        in_specs=[
            pl.BlockSpec((BE, EDGE_IN), lambda i: (i, 0)),
            pl.BlockSpec((BE, HS), lambda i: (i, 0)),
            pl.BlockSpec((EDGE_IN, EH), lambda i: (0, 0)),
            pl.BlockSpec((HS, EH), lambda i: (0, 0)),
            pl.BlockSpec((1, EH), lambda i: (0, 0)),
        ],
        out_specs=pl.BlockSpec((BE, GH), lambda i: (i, 0)),
        out_shape=jax.ShapeDtypeStruct((EP, GH), jnp.float32),
    )(opt, hid, wa, wb, b)


def _tc_edge_mlp(gath, eh, wc, b1, w2, b2, g, bl):
    nb = E // BE

    def body(g_sum_r, e_r, wc_r, b1_r, w2_r, b2_r, g_r, bl_r, out_r):
        e_v = e_r[...][:, :EH]
        h = g_sum_r[...] + _dot(e_v, wc_r[...]) + b1_r[...]
        m = _dot(_gelu(h), w2_r[...]) + b2_r[...]
        x = _ln_apply(e_v + m, g_r[...], bl_r[...])
        out_r[...] = jnp.concatenate([x, x], axis=-1)

    return pl.pallas_call(
        body,
        grid=(nb,),
        in_specs=[
            pl.BlockSpec((BE, GH), lambda i: (i, 0)),
            pl.BlockSpec((BE, GH), lambda i: (i, 0)),
            pl.BlockSpec((EH, GH), lambda i: (0, 0)),
            pl.BlockSpec((1, GH), lambda i: (0, 0)),
            pl.BlockSpec((GH, EH), lambda i: (0, 0)),
            pl.BlockSpec((1, EH), lambda i: (0, 0)),
            pl.BlockSpec((1, EH), lambda i: (0, 0)),
            pl.BlockSpec((1, EH), lambda i: (0, 0)),
        ],
        out_specs=pl.BlockSpec((BE, GH), lambda i: (i, 0)),
        out_shape=jax.ShapeDtypeStruct((EP, GH), jnp.float32),
    )(gath, eh, wc, b1, w2, b2, g, bl)


def _tc_deginv(degp):
    def body(d_r, o_r):
        d = d_r[...]
        o_r[...] = 1.0 / jnp.maximum(d[:N, :NH] + d[NP_:NP_ + N, :NH], 1.0)
    return pl.pallas_call(
        body, out_shape=jax.ShapeDtypeStruct((N, NH), jnp.float32),
    )(degp)


def _tc_node_mlp(nh, aggp, dinv, wd, wf, b1, w2, b2, g, bl, wa, wb):
    def body(nh_r, a_r, di_r, wd_r, wf_r, b1_r, w2_r, b2_r, g_r, bl_r,
             wa_r, wb_r, out_r, p_r):
        nh_v = nh_r[...]
        a = a_r[...]
        agg = (a[:N, :NH] + a[NP_:NP_ + N, :NH]) * di_r[...]
        h = _gelu(_dot(nh_v, wd_r[...]) + _dot(agg, wf_r[...]) + b1_r[...])
        u = _dot(h, w2_r[...]) + b2_r[...]
        nh_new = _ln_apply(nh_v + u, g_r[...], bl_r[...])
        out_r[...] = nh_new
        p_r[...] = _proj_table(nh_new, wa_r[...], wb_r[...])
    return pl.pallas_call(
        body,
        out_shape=[jax.ShapeDtypeStruct((N, NH), jnp.float32),
                   jax.ShapeDtypeStruct((2 * N, GH), jnp.float32)],
    )(nh, aggp, dinv, wd, wf, b1, w2, b2, g, bl, wa, wb)


def _tc_heads(eh, hid, wh1, bh1, wh2, bh2, wx, whh, bx, bhh):
    nb = E // BE

    def body(e_r, h_r, wh1_r, bh1_r, wh2_r, bh2_r, wx_r, whh_r, bx_r, bhh_r,
             u_r, nh_r):
        e_v = e_r[...][:, :EH]
        h_v = h_r[...]
        t = _gelu(_dot(e_v, wh1_r[...]) + bh1_r[...])
        u_r[...] = (_dot(t, wh2_r[...]) + bh2_r[...]) * SCALE
        gx = _dot(e_v, wx_r[...]) + bx_r[...]
        gh = _dot(h_v, whh_r[...]) + bhh_r[...]
        r = jax.nn.sigmoid(gx[:, :HS] + gh[:, :HS])
        z = jax.nn.sigmoid(gx[:, HS:2 * HS] + gh[:, HS:2 * HS])
        n = jnp.tanh(gx[:, 2 * HS:] + r * gh[:, 2 * HS:])
        nh_r[...] = (1.0 - z) * n + z * h_v

    return pl.pallas_call(
        body,
        grid=(nb,),
        in_specs=[
            pl.BlockSpec((BE, GH), lambda i: (i, 0)),
            pl.BlockSpec((BE, HS), lambda i: (i, 0)),
            pl.BlockSpec((EH, GH), lambda i: (0, 0)),
            pl.BlockSpec((1, GH), lambda i: (0, 0)),
            pl.BlockSpec((GH, 1), lambda i: (0, 0)),
            pl.BlockSpec((1, 1), lambda i: (0, 0)),
            pl.BlockSpec((EH, 3 * HS), lambda i: (0, 0)),
            pl.BlockSpec((HS, 3 * HS), lambda i: (0, 0)),
            pl.BlockSpec((1, 3 * HS), lambda i: (0, 0)),
            pl.BlockSpec((1, 3 * HS), lambda i: (0, 0)),
        ],
        out_specs=[
            pl.BlockSpec((BE, 1), lambda i: (i, 0)),
            pl.BlockSpec((BE, HS), lambda i: (i, 0)),
        ],
        out_shape=[
            jax.ShapeDtypeStruct((E, 1), jnp.float32),
            jax.ShapeDtypeStruct((E, HS), jnp.float32),
        ],
    )(eh, hid, wh1, bh1, wh2, bh2, wx, whh, bx, bhh)


def kernel(node_features, edge_index, optimizer_features, hidden_state, params):
    p = params
    src = edge_index[0]
    dst = edge_index[1]
    pad_n = EP - E
    idx_src = jnp.concatenate(
        [src, jnp.zeros((pad_n,), jnp.int32)]).reshape(EP // CH, CH)
    idx_dst = jnp.concatenate(
        [dst + N, jnp.full((pad_n,), N, jnp.int32)]).reshape(EP // CH, CH)
    dst_pad = jnp.concatenate(
        [dst, jnp.full((pad_n,), DUMP, jnp.int32)]).reshape(EP // CH, CH)
    zeros_acc = jnp.zeros((NP_, GH), jnp.float32)
    ones_vals = jnp.ones((2 * CH, GH), jnp.float32)

    gather_f = _build_gather()
    segsum_f = _build_segsum(const_ones=False)
    degree_f = _build_segsum(const_ones=True)

    r1 = lambda v: v.reshape(1, -1)
    lyrs = p["layers"]
    splits = []
    for lyr in lyrs:
        splits.append((lyr["We1"][:NH], lyr["We1"][NH:2 * NH],
                       lyr["We1"][2 * NH:], lyr["Wn1"][:NH], lyr["Wn1"][NH:]))

    node_h, ptab = _tc_node_proj(node_features, p["Wnp"], r1(p["bnp"]),
                                 splits[0][0], splits[0][1])
    edge_h = _tc_edge_proj(optimizer_features, hidden_state,
                           p["Wep"][:EDGE_IN], p["Wep"][EDGE_IN:], r1(p["bep"]))

    degp = degree_f(ones_vals, dst_pad, zeros_acc)
    dinv = _tc_deginv(degp)

    n_rounds = L * R
    for t in range(n_rounds):
        lyr = lyrs[t // R]
        wa, wb, wc, wd, wf = splits[t // R]
        na, nb_ = splits[min(t + 1, n_rounds - 1) // R][:2]
        gath = gather_f(ptab, idx_src, idx_dst)
        edge_h = _tc_edge_mlp(gath, edge_h, wc, r1(lyr["be1"]),
                              lyr["We2"], r1(lyr["be2"]),
                              r1(lyr["ge"]), r1(lyr["be_ln"]))
        aggp = segsum_f(edge_h, dst_pad, zeros_acc)
        node_h, ptab = _tc_node_mlp(node_h, aggp, dinv, wd, wf, r1(lyr["bn1"]),
                                    lyr["Wn2"], r1(lyr["bn2"]),
                                    r1(lyr["gn"]), r1(lyr["bn_ln"]), na, nb_)

    updates, new_hidden = _tc_heads(
        edge_h, hidden_state, p["Wh1"], r1(p["bh1"]), p["Wh2"], r1(p["bh2"]),
        p["Wx"], p["Whh"], r1(p["bx"]), r1(p["bhh"]))
    return updates, new_hidden


# speedup vs baseline: 2.3842x; 1.0005x over previous
import functools

import jax
import jax.numpy as jnp
from jax import lax
from jax.experimental import pallas as pl
from jax.experimental.pallas import tpu as pltpu
from jax.experimental.pallas import tpu_sc as plsc

N = 10000
E = 320000
NODE_IN = 128
EDGE_IN = 16
NH = 64
EH = 64
GH = 128
L = 3
R = 2
HS = 32
SCALE = 1e-3

NC, NS = 2, 16
NW = NC * NS
CH = 128
RPC = 8
CHUNK = CH * RPC
HALF = CHUNK // 2
BE = 2000

NP_ = 10240
RZ = NP_ // NS
DUMP = N + 8
EP = ((E + CHUNK - 1) // CHUNK) * CHUNK
G2E = 2 * E


@functools.lru_cache(maxsize=None)
def _mesh():
    return plsc.VectorSubcoreMesh(
        core_axis_name="c", subcore_axis_name="s",
        num_cores=NC, num_subcores=NS)


def _wid():
    return lax.axis_index("s") * NC + lax.axis_index("c")


def _build_gather(interpret=False):
    n_chunks = EP // CHUNK
    k_max = (n_chunks + NW - 1) // NW
    QR = 2 * CH
    NQ = CHUNK // QR

    @functools.partial(
        pl.kernel, mesh=_mesh(), interpret=interpret,
        out_type=jax.ShapeDtypeStruct((EP, GH), jnp.float32),
        scratch_types=[
            pltpu.VMEM((RPC, CH), jnp.int32),
            pltpu.VMEM((RPC, CH), jnp.int32),
            pltpu.VMEM((QR, GH), jnp.float32),
            pltpu.VMEM((QR, GH), jnp.float32),
            pltpu.SemaphoreType.DMA,
            pltpu.SemaphoreType.DMA,
            pltpu.SemaphoreType.DMA,
        ],
    )
    def gather_k(table_hbm, idxs_hbm, idxd_hbm, out_hbm, idxs_v, idxd_v,
                 rows0_v, rows1_v, semg, semo0, semo1):
        wid = _wid()
        bufs = (rows0_v, rows1_v)
        sems = (semo0, semo1)

        def body(k, carry):
            chunk = wid + k * NW

            @pl.when(chunk < n_chunks)
            def _():
                rbase = pl.multiple_of(chunk * RPC, RPC)
                pltpu.sync_copy(idxs_hbm.at[pl.ds(rbase, RPC)], idxs_v)
                pltpu.sync_copy(idxd_hbm.at[pl.ds(rbase, RPC)], idxd_v)
                for q in range(NQ):
                    rows_v = bufs[q % 2]
                    semo = sems[q % 2]
                    drain = lambda: pltpu.make_async_copy(
                        table_hbm.at[pl.ds(0, QR)], rows_v, semo).wait()
                    if q < 2:
                        pl.when(k > 0)(drain)
                    else:
                        drain()
                    descs = []
                    for j in range(QR // CH):
                        descs.append(pltpu.async_copy(
                            table_hbm.at[idxs_v.at[q * (QR // CH) + j]],
                            rows_v.at[pl.ds(j * CH, CH)], semg))
                    for d in descs:
                        d.wait()
                    descs = []
                    for j in range(QR // CH):
                        descs.append(pltpu.async_copy(
                            table_hbm.at[idxd_v.at[q * (QR // CH) + j]],
                            rows_v.at[pl.ds(j * CH, CH)], semg, add=True))
                    for d in descs:
                        d.wait()
                    obase = pl.multiple_of(chunk * CHUNK + q * QR, QR)
                    pltpu.async_copy(rows_v, out_hbm.at[pl.ds(obase, QR)],
                                     semo)

            return carry

        lax.fori_loop(0, k_max, body, 0)
        pltpu.make_async_copy(table_hbm.at[pl.ds(0, QR)], rows0_v,
                              semo0).wait()
        pltpu.make_async_copy(table_hbm.at[pl.ds(0, QR)], rows1_v,
                              semo1).wait()

    return gather_k


def _build_segsum(const_ones, interpret=False):
    n_chunks = EP // CHUNK
    k_max = (n_chunks + NW - 1) // NW
    q_rows = 2 * CH

    @functools.partial(
        pl.kernel, mesh=_mesh(), interpret=interpret,
        out_type=jax.ShapeDtypeStruct((NC * NP_, GH), jnp.float32),
        scratch_types=[
            pltpu.VMEM((RPC, CH), jnp.int32),
            pltpu.VMEM((q_rows, GH), jnp.float32),
            pltpu.VMEM_SHARED((NP_, GH), jnp.float32),
        ],
    )
    def segsum_k(vals_hbm, idx_hbm, zeros_hbm, out_hbm, idx_v, rows_v, acc_sh):
        cid = lax.axis_index("c")
        sid = lax.axis_index("s")
        wid = _wid()
        zbase = pl.multiple_of(sid * RZ, RZ)
        pltpu.sync_copy(zeros_hbm.at[pl.ds(zbase, RZ)],
                        acc_sh.at[pl.ds(zbase, RZ)])
        if const_ones:
            pltpu.sync_copy(vals_hbm, rows_v)
        plsc.subcore_barrier()

        def body(k, carry):
            chunk = wid + k * NW

            @pl.when(chunk < n_chunks)
            def _():
                rbase = pl.multiple_of(chunk * RPC, RPC)
                pltpu.sync_copy(idx_hbm.at[pl.ds(rbase, RPC)], idx_v)
                for q in range(CHUNK // q_rows):
                    if not const_ones:
                        ebase = pl.multiple_of(
                            chunk * CHUNK + q * q_rows, q_rows)
                        pltpu.sync_copy(vals_hbm.at[pl.ds(ebase, q_rows)],
                                        rows_v)
                    for j in range(q_rows // CH):
                        pltpu.sync_copy(
                            rows_v.at[pl.ds(j * CH, CH)],
                            acc_sh.at[idx_v.at[q * (q_rows // CH) + j]],
                            add=True)

            return carry

        lax.fori_loop(0, k_max, body, 0)
        plsc.subcore_barrier()
        obase = pl.multiple_of(cid * NP_ + sid * RZ, RZ)
        pltpu.sync_copy(acc_sh.at[pl.ds(zbase, RZ)],
                        out_hbm.at[pl.ds(obase, RZ)])

    return segsum_k


def _gelu(x):
    return jax.nn.gelu(x)


def _dot(a, b):
    return jnp.dot(a, b, preferred_element_type=jnp.float32)


def _ln_apply(x, g, b):
    mu = jnp.mean(x, axis=-1, keepdims=True)
    var = jnp.mean((x - mu) ** 2, axis=-1, keepdims=True)
    return (x - mu) / jnp.sqrt(var + 1e-5) * g + b


def _proj_table(nh, wa, wb):
    return jnp.concatenate([_dot(nh, wa), _dot(nh, wb)], axis=0)


def _tc_node_proj(nf, w, b, wa, wb):
    def body(nf_r, w_r, b_r, wa_r, wb_r, o_r, p_r):
        nh = _gelu(_dot(nf_r[...], w_r[...]) + b_r[...])
        o_r[...] = nh
        p_r[...] = _proj_table(nh, wa_r[...], wb_r[...])
    return pl.pallas_call(
        body,
        out_shape=[jax.ShapeDtypeStruct((N, NH), jnp.float32),
                   jax.ShapeDtypeStruct((2 * N, GH), jnp.float32)],
    )(nf, w, b, wa, wb)


def _tc_edge_proj(opt, hid, wa, wb, b):
    def body(o_r, h_r, wa_r, wb_r, b_r, out_r):
        x = _gelu(_dot(o_r[...], wa_r[...]) +
                  _dot(h_r[...], wb_r[...]) + b_r[...])
        out_r[...] = jnp.concatenate([x, x], axis=-1)
    nb = E // BE
    return pl.pallas_call(
        body,
        grid=(nb,),
        in_specs=[
            pl.BlockSpec((BE, EDGE_IN), lambda i: (i, 0)),
            pl.BlockSpec((BE, HS), lambda i: (i, 0)),
            pl.BlockSpec((EDGE_IN, EH), lambda i: (0, 0)),
            pl.BlockSpec((HS, EH), lambda i: (0, 0)),
            pl.BlockSpec((1, EH), lambda i: (0, 0)),
        ],
        out_specs=pl.BlockSpec((BE, GH), lambda i: (i, 0)),
        out_shape=jax.ShapeDtypeStruct((EP, GH), jnp.float32),
    )(opt, hid, wa, wb, b)


def _tc_edge_mlp(gath, eh, wc, b1, w2, b2, g, bl):
    nb = E // BE

    def body(g_sum_r, e_r, wc_r, b1_r, w2_r, b2_r, g_r, bl_r, out_r):
        e_v = e_r[...][:, :EH]
        h = g_sum_r[...] + _dot(e_v, wc_r[...]) + b1_r[...]
        m = _dot(_gelu(h), w2_r[...]) + b2_r[...]
        x = _ln_apply(e_v + m, g_r[...], bl_r[...])
        out_r[...] = jnp.concatenate([x, x], axis=-1)

    return pl.pallas_call(
        body,
        grid=(nb,),
        in_specs=[
            pl.BlockSpec((BE, GH), lambda i: (i, 0)),
            pl.BlockSpec((BE, GH), lambda i: (i, 0)),
            pl.BlockSpec((EH, GH), lambda i: (0, 0)),
            pl.BlockSpec((1, GH), lambda i: (0, 0)),
            pl.BlockSpec((GH, EH), lambda i: (0, 0)),
            pl.BlockSpec((1, EH), lambda i: (0, 0)),
            pl.BlockSpec((1, EH), lambda i: (0, 0)),
            pl.BlockSpec((1, EH), lambda i: (0, 0)),
        ],
        out_specs=pl.BlockSpec((BE, GH), lambda i: (i, 0)),
        out_shape=jax.ShapeDtypeStruct((EP, GH), jnp.float32),
    )(gath, eh, wc, b1, w2, b2, g, bl)


def _tc_deginv(degp):
    def body(d_r, o_r):
        d = d_r[...]
        o_r[...] = 1.0 / jnp.maximum(d[:N, :NH] + d[NP_:NP_ + N, :NH], 1.0)
    return pl.pallas_call(
        body, out_shape=jax.ShapeDtypeStruct((N, NH), jnp.float32),
    )(degp)


def _tc_node_mlp(nh, aggp, dinv, wd, wf, b1, w2, b2, g, bl, wa, wb):
    def body(nh_r, a_r, di_r, wd_r, wf_r, b1_r, w2_r, b2_r, g_r, bl_r,
             wa_r, wb_r, out_r, p_r):
        nh_v = nh_r[...]
        a = a_r[...]
        agg = (a[:N, :NH] + a[NP_:NP_ + N, :NH]) * di_r[...]
        h = _gelu(_dot(nh_v, wd_r[...]) + _dot(agg, wf_r[...]) + b1_r[...])
        u = _dot(h, w2_r[...]) + b2_r[...]
        nh_new = _ln_apply(nh_v + u, g_r[...], bl_r[...])
        out_r[...] = nh_new
        p_r[...] = _proj_table(nh_new, wa_r[...], wb_r[...])
    return pl.pallas_call(
        body,
        out_shape=[jax.ShapeDtypeStruct((N, NH), jnp.float32),
                   jax.ShapeDtypeStruct((2 * N, GH), jnp.float32)],
    )(nh, aggp, dinv, wd, wf, b1, w2, b2, g, bl, wa, wb)


def _tc_heads(eh, hid, wh1, bh1, wh2, bh2, wx, whh, bx, bhh):
    nb = E // BE

    def body(e_r, h_r, wh1_r, bh1_r, wh2_r, bh2_r, wx_r, whh_r, bx_r, bhh_r,
             u_r, nh_r):
        e_v = e_r[...][:, :EH]
        h_v = h_r[...]
        t = _gelu(_dot(e_v, wh1_r[...]) + bh1_r[...])
        u_r[...] = (_dot(t, wh2_r[...]) + bh2_r[...]) * SCALE
        gx = _dot(e_v, wx_r[...]) + bx_r[...]
        gh = _dot(h_v, whh_r[...]) + bhh_r[...]
        r = jax.nn.sigmoid(gx[:, :HS] + gh[:, :HS])
        z = jax.nn.sigmoid(gx[:, HS:2 * HS] + gh[:, HS:2 * HS])
        n = jnp.tanh(gx[:, 2 * HS:] + r * gh[:, 2 * HS:])
        nh_r[...] = (1.0 - z) * n + z * h_v

    return pl.pallas_call(
        body,
        grid=(nb,),
        in_specs=[
            pl.BlockSpec((BE, GH), lambda i: (i, 0)),
            pl.BlockSpec((BE, HS), lambda i: (i, 0)),
            pl.BlockSpec((EH, GH), lambda i: (0, 0)),
            pl.BlockSpec((1, GH), lambda i: (0, 0)),
            pl.BlockSpec((GH, 1), lambda i: (0, 0)),
            pl.BlockSpec((1, 1), lambda i: (0, 0)),
            pl.BlockSpec((EH, 3 * HS), lambda i: (0, 0)),
            pl.BlockSpec((HS, 3 * HS), lambda i: (0, 0)),
            pl.BlockSpec((1, 3 * HS), lambda i: (0, 0)),
            pl.BlockSpec((1, 3 * HS), lambda i: (0, 0)),
        ],
        out_specs=[
            pl.BlockSpec((BE, 1), lambda i: (i, 0)),
            pl.BlockSpec((BE, HS), lambda i: (i, 0)),
        ],
        out_shape=[
            jax.ShapeDtypeStruct((E, 1), jnp.float32),
            jax.ShapeDtypeStruct((E, HS), jnp.float32),
        ],
    )(eh, hid, wh1, bh1, wh2, bh2, wx, whh, bx, bhh)


def kernel(node_features, edge_index, optimizer_features, hidden_state, params):
    p = params
    src = edge_index[0]
    dst = edge_index[1]
    pad_n = EP - E
    idx_src = jnp.concatenate(
        [src, jnp.zeros((pad_n,), jnp.int32)]).reshape(EP // CH, CH)
    idx_dst = jnp.concatenate(
        [dst + N, jnp.full((pad_n,), N, jnp.int32)]).reshape(EP // CH, CH)
    dst_pad = jnp.concatenate(
        [dst, jnp.full((pad_n,), DUMP, jnp.int32)]).reshape(EP // CH, CH)
    zeros_acc = jnp.zeros((NP_, GH), jnp.float32)
    ones_vals = jnp.ones((2 * CH, GH), jnp.float32)

    gather_f = _build_gather()
    segsum_f = _build_segsum(const_ones=False)
    degree_f = _build_segsum(const_ones=True)

    r1 = lambda v: v.reshape(1, -1)
    lyrs = p["layers"]
    splits = []
    for lyr in lyrs:
        splits.append((lyr["We1"][:NH], lyr["We1"][NH:2 * NH],
                       lyr["We1"][2 * NH:], lyr["Wn1"][:NH], lyr["Wn1"][NH:]))

    node_h, ptab = _tc_node_proj(node_features, p["Wnp"], r1(p["bnp"]),
                                 splits[0][0], splits[0][1])
    edge_h = _tc_edge_proj(optimizer_features, hidden_state,
                           p["Wep"][:EDGE_IN], p["Wep"][EDGE_IN:], r1(p["bep"]))

    degp = degree_f(ones_vals, dst_pad, zeros_acc)
    dinv = _tc_deginv(degp)

    n_rounds = L * R
    for t in range(n_rounds):
        lyr = lyrs[t // R]
        wa, wb, wc, wd, wf = splits[t // R]
        na, nb_ = splits[min(t + 1, n_rounds - 1) // R][:2]
        gath = gather_f(ptab, idx_src, idx_dst)
        edge_h = _tc_edge_mlp(gath, edge_h, wc, r1(lyr["be1"]),
                              lyr["We2"], r1(lyr["be2"]),
                              r1(lyr["ge"]), r1(lyr["be_ln"]))
        aggp = segsum_f(edge_h, dst_pad, zeros_acc)
        node_h, ptab = _tc_node_mlp(node_h, aggp, dinv, wd, wf, r1(lyr["bn1"]),
                                    lyr["Wn2"], r1(lyr["bn2"]),
                                    r1(lyr["gn"]), r1(lyr["bn_ln"]), na, nb_)

    updates, new_hidden = _tc_heads(
        edge_h, hidden_state, p["Wh1"], r1(p["bh1"]), p["Wh2"], r1(p["bh2"]),
        p["Wx"], p["Whh"], r1(p["bx"]), r1(p["bhh"]))
    return updates, new_hidden


# speedup vs baseline: 2.5315x; 1.0618x over previous
import functools

import jax
import jax.numpy as jnp
from jax import lax
from jax.experimental import pallas as pl
from jax.experimental.pallas import tpu as pltpu
from jax.experimental.pallas import tpu_sc as plsc

N = 10000
E = 320000
NODE_IN = 128
EDGE_IN = 16
NH = 64
EH = 64
GH = 128
L = 3
R = 2
HS = 32
SCALE = 1e-3

NC, NS = 2, 16
NW = NC * NS
CH = 128
RPC = 8
CHUNK = CH * RPC
HALF = CHUNK // 2
BE = 2000

NP_ = 10240
RZ = NP_ // NS
DUMP = N + 8
EP = ((E + CHUNK - 1) // CHUNK) * CHUNK
G2E = 2 * E


@functools.lru_cache(maxsize=None)
def _mesh():
    return plsc.VectorSubcoreMesh(
        core_axis_name="c", subcore_axis_name="s",
        num_cores=NC, num_subcores=NS)


def _wid():
    return lax.axis_index("s") * NC + lax.axis_index("c")


def _build_gather(interpret=False):
    n_chunks = EP // CHUNK
    k_max = (n_chunks + NW - 1) // NW
    NB = 4

    @functools.partial(
        pl.kernel, mesh=_mesh(), interpret=interpret,
        out_type=jax.ShapeDtypeStruct((EP, GH), jnp.float32),
        scratch_types=[
            pltpu.VMEM((RPC, CH), jnp.int32),
            pltpu.VMEM((RPC, CH), jnp.int32),
            [pltpu.VMEM((CH, GH), jnp.float32) for _ in range(NB)],
            [pltpu.SemaphoreType.DMA for _ in range(NB)],
            pltpu.SemaphoreType.DMA,
            [pltpu.SemaphoreType.DMA for _ in range(NB)],
        ],
    )
    def gather_k(table_hbm, idxs_hbm, idxd_hbm, out_hbm, idxs_v, idxd_v,
                 bufs, semb, sema, semo):
        wid = _wid()

        def drain_out(b):
            pltpu.make_async_copy(table_hbm.at[pl.ds(0, CH)], bufs[b],
                                  semo[b]).wait()

        def base_op(u):
            return pltpu.async_copy(table_hbm.at[idxs_v.at[u]],
                                    bufs[u % NB], semb[u % NB])

        def body(k, carry):
            chunk = wid + k * NW

            @pl.when(chunk < n_chunks)
            def _():
                rbase = pl.multiple_of(chunk * RPC, RPC)
                pltpu.sync_copy(idxs_hbm.at[pl.ds(rbase, RPC)], idxs_v)
                pltpu.sync_copy(idxd_hbm.at[pl.ds(rbase, RPC)], idxd_v)
                base_d = [None] * RPC
                for u in range(2):
                    pl.when(k > 0)(functools.partial(drain_out, u))
                    base_d[u] = base_op(u)
                for u in range(RPC):
                    base_d[u].wait()
                    add_d = pltpu.async_copy(
                        table_hbm.at[idxd_v.at[u]], bufs[u % NB], sema,
                        add=True)
                    v = u + 2
                    if v < RPC:
                        if v < NB:
                            pl.when(k > 0)(functools.partial(drain_out, v))
                        else:
                            drain_out(v % NB)
                        base_d[v] = base_op(v)
                    add_d.wait()
                    obase = pl.multiple_of(chunk * CHUNK + u * CH, CH)
                    pltpu.async_copy(bufs[u % NB],
                                     out_hbm.at[pl.ds(obase, CH)],
                                     semo[u % NB])

            return carry

        lax.fori_loop(0, k_max, body, 0)
        for b in range(NB):
            drain_out(b)

    return gather_k


def _build_segsum(const_ones, interpret=False):
    n_chunks = EP // CHUNK
    k_max = (n_chunks + NW - 1) // NW

    @functools.partial(
        pl.kernel, mesh=_mesh(), interpret=interpret,
        out_type=jax.ShapeDtypeStruct((NC * NP_, GH), jnp.float32),
        scratch_types=[
            pltpu.VMEM((RPC, CH), jnp.int32),
            [pltpu.VMEM((CH, GH), jnp.float32) for _ in range(2)],
            [pltpu.SemaphoreType.DMA for _ in range(2)],
            [pltpu.SemaphoreType.DMA for _ in range(2)],
            pltpu.VMEM_SHARED((NP_, GH), jnp.float32),
        ],
    )
    def segsum_k(vals_hbm, idx_hbm, zeros_hbm, out_hbm, idx_v, bufs, semv,
                 sems, acc_sh):
        cid = lax.axis_index("c")
        sid = lax.axis_index("s")
        wid = _wid()
        zbase = pl.multiple_of(sid * RZ, RZ)
        pltpu.sync_copy(zeros_hbm.at[pl.ds(zbase, RZ)],
                        acc_sh.at[pl.ds(zbase, RZ)])
        if const_ones:
            pltpu.sync_copy(vals_hbm, bufs[0])
        plsc.subcore_barrier()

        def drain_add(b):
            pltpu.make_async_copy(vals_hbm.at[pl.ds(0, CH)], bufs[b],
                                  sems[b]).wait()

        def vals_op(chunk, u):
            ebase = pl.multiple_of(chunk * CHUNK + u * CH, CH)
            return pltpu.async_copy(vals_hbm.at[pl.ds(ebase, CH)],
                                    bufs[u % 2], semv[u % 2])

        def body(k, carry):
            chunk = wid + k * NW

            @pl.when(chunk < n_chunks)
            def _():
                rbase = pl.multiple_of(chunk * RPC, RPC)
                pltpu.sync_copy(idx_hbm.at[pl.ds(rbase, RPC)], idx_v)
                if const_ones:
                    descs = [pltpu.async_copy(bufs[0],
                                              acc_sh.at[idx_v.at[u]],
                                              sems[0], add=True)
                             for u in range(RPC)]
                    for d in descs:
                        d.wait()
                else:
                    pl.when(k > 0)(functools.partial(drain_add, 0))
                    vals_d = [None] * RPC
                    vals_d[0] = vals_op(chunk, 0)
                    for u in range(RPC):
                        v = u + 1
                        if v < RPC:
                            if v == 1:
                                pl.when(k > 0)(
                                    functools.partial(drain_add, 1))
                            else:
                                drain_add(v % 2)
                            vals_d[v] = vals_op(chunk, v)
                        vals_d[u].wait()
                        pltpu.async_copy(bufs[u % 2],
                                         acc_sh.at[idx_v.at[u]],
                                         sems[u % 2], add=True)

            return carry

        lax.fori_loop(0, k_max, body, 0)
        if not const_ones:
            drain_add(0)
            drain_add(1)
        plsc.subcore_barrier()
        obase = pl.multiple_of(cid * NP_ + sid * RZ, RZ)
        pltpu.sync_copy(acc_sh.at[pl.ds(zbase, RZ)],
                        out_hbm.at[pl.ds(obase, RZ)])

    return segsum_k


def _gelu(x):
    return jax.nn.gelu(x)


def _dot(a, b):
    return jnp.dot(a, b, preferred_element_type=jnp.float32)


def _ln_apply(x, g, b):
    mu = jnp.mean(x, axis=-1, keepdims=True)
    var = jnp.mean((x - mu) ** 2, axis=-1, keepdims=True)
    return (x - mu) / jnp.sqrt(var + 1e-5) * g + b


def _proj_table(nh, wa, wb):
    return jnp.concatenate([_dot(nh, wa), _dot(nh, wb)], axis=0)


def _tc_node_proj(nf, w, b, wa, wb):
    def body(nf_r, w_r, b_r, wa_r, wb_r, o_r, p_r):
        nh = _gelu(_dot(nf_r[...], w_r[...]) + b_r[...])
        o_r[...] = nh
        p_r[...] = _proj_table(nh, wa_r[...], wb_r[...])
    return pl.pallas_call(
        body,
        out_shape=[jax.ShapeDtypeStruct((N, NH), jnp.float32),
                   jax.ShapeDtypeStruct((2 * N, GH), jnp.float32)],
    )(nf, w, b, wa, wb)


def _tc_edge_proj(opt, hid, wa, wb, b):
    def body(o_r, h_r, wa_r, wb_r, b_r, out_r):
        x = _gelu(_dot(o_r[...], wa_r[...]) +
                  _dot(h_r[...], wb_r[...]) + b_r[...])
        out_r[...] = jnp.concatenate([x, x], axis=-1)
    nb = E // BE
    return pl.pallas_call(
        body,
        grid=(nb,),
        in_specs=[
            pl.BlockSpec((BE, EDGE_IN), lambda i: (i, 0)),
            pl.BlockSpec((BE, HS), lambda i: (i, 0)),
            pl.BlockSpec((EDGE_IN, EH), lambda i: (0, 0)),
            pl.BlockSpec((HS, EH), lambda i: (0, 0)),
            pl.BlockSpec((1, EH), lambda i: (0, 0)),
        ],
        out_specs=pl.BlockSpec((BE, GH), lambda i: (i, 0)),
        out_shape=jax.ShapeDtypeStruct((EP, GH), jnp.float32),
    )(opt, hid, wa, wb, b)


def _tc_edge_mlp(gath, eh, wc, b1, w2, b2, g, bl):
    nb = E // BE

    def body(g_sum_r, e_r, wc_r, b1_r, w2_r, b2_r, g_r, bl_r, out_r):
        e_v = e_r[...][:, :EH]
        h = g_sum_r[...] + _dot(e_v, wc_r[...]) + b1_r[...]
        m = _dot(_gelu(h), w2_r[...]) + b2_r[...]
        x = _ln_apply(e_v + m, g_r[...], bl_r[...])
        out_r[...] = jnp.concatenate([x, x], axis=-1)

    return pl.pallas_call(
        body,
        grid=(nb,),
        in_specs=[
            pl.BlockSpec((BE, GH), lambda i: (i, 0)),
            pl.BlockSpec((BE, GH), lambda i: (i, 0)),
            pl.BlockSpec((EH, GH), lambda i: (0, 0)),
            pl.BlockSpec((1, GH), lambda i: (0, 0)),
            pl.BlockSpec((GH, EH), lambda i: (0, 0)),
            pl.BlockSpec((1, EH), lambda i: (0, 0)),
            pl.BlockSpec((1, EH), lambda i: (0, 0)),
            pl.BlockSpec((1, EH), lambda i: (0, 0)),
        ],
        out_specs=pl.BlockSpec((BE, GH), lambda i: (i, 0)),
        out_shape=jax.ShapeDtypeStruct((EP, GH), jnp.float32),
    )(gath, eh, wc, b1, w2, b2, g, bl)


def _tc_deginv(degp):
    def body(d_r, o_r):
        d = d_r[...]
        o_r[...] = 1.0 / jnp.maximum(d[:N, :NH] + d[NP_:NP_ + N, :NH], 1.0)
    return pl.pallas_call(
        body, out_shape=jax.ShapeDtypeStruct((N, NH), jnp.float32),
    )(degp)


def _tc_node_mlp(nh, aggp, dinv, wd, wf, b1, w2, b2, g, bl, wa, wb):
    def body(nh_r, a_r, di_r, wd_r, wf_r, b1_r, w2_r, b2_r, g_r, bl_r,
             wa_r, wb_r, out_r, p_r):
        nh_v = nh_r[...]
        a = a_r[...]
        agg = (a[:N, :NH] + a[NP_:NP_ + N, :NH]) * di_r[...]
        h = _gelu(_dot(nh_v, wd_r[...]) + _dot(agg, wf_r[...]) + b1_r[...])
        u = _dot(h, w2_r[...]) + b2_r[...]
        nh_new = _ln_apply(nh_v + u, g_r[...], bl_r[...])
        out_r[...] = nh_new
        p_r[...] = _proj_table(nh_new, wa_r[...], wb_r[...])
    return pl.pallas_call(
        body,
        out_shape=[jax.ShapeDtypeStruct((N, NH), jnp.float32),
                   jax.ShapeDtypeStruct((2 * N, GH), jnp.float32)],
    )(nh, aggp, dinv, wd, wf, b1, w2, b2, g, bl, wa, wb)


def _tc_heads(eh, hid, wh1, bh1, wh2, bh2, wx, whh, bx, bhh):
    nb = E // BE

    def body(e_r, h_r, wh1_r, bh1_r, wh2_r, bh2_r, wx_r, whh_r, bx_r, bhh_r,
             u_r, nh_r):
        e_v = e_r[...][:, :EH]
        h_v = h_r[...]
        t = _gelu(_dot(e_v, wh1_r[...]) + bh1_r[...])
        u_r[...] = (_dot(t, wh2_r[...]) + bh2_r[...]) * SCALE
        gx = _dot(e_v, wx_r[...]) + bx_r[...]
        gh = _dot(h_v, whh_r[...]) + bhh_r[...]
        r = jax.nn.sigmoid(gx[:, :HS] + gh[:, :HS])
        z = jax.nn.sigmoid(gx[:, HS:2 * HS] + gh[:, HS:2 * HS])
        n = jnp.tanh(gx[:, 2 * HS:] + r * gh[:, 2 * HS:])
        nh_r[...] = (1.0 - z) * n + z * h_v

    return pl.pallas_call(
        body,
        grid=(nb,),
        in_specs=[
            pl.BlockSpec((BE, GH), lambda i: (i, 0)),
            pl.BlockSpec((BE, HS), lambda i: (i, 0)),
            pl.BlockSpec((EH, GH), lambda i: (0, 0)),
            pl.BlockSpec((1, GH), lambda i: (0, 0)),
            pl.BlockSpec((GH, 1), lambda i: (0, 0)),
            pl.BlockSpec((1, 1), lambda i: (0, 0)),
            pl.BlockSpec((EH, 3 * HS), lambda i: (0, 0)),
            pl.BlockSpec((HS, 3 * HS), lambda i: (0, 0)),
            pl.BlockSpec((1, 3 * HS), lambda i: (0, 0)),
            pl.BlockSpec((1, 3 * HS), lambda i: (0, 0)),
        ],
        out_specs=[
            pl.BlockSpec((BE, 1), lambda i: (i, 0)),
            pl.BlockSpec((BE, HS), lambda i: (i, 0)),
        ],
        out_shape=[
            jax.ShapeDtypeStruct((E, 1), jnp.float32),
            jax.ShapeDtypeStruct((E, HS), jnp.float32),
        ],
    )(eh, hid, wh1, bh1, wh2, bh2, wx, whh, bx, bhh)


def kernel(node_features, edge_index, optimizer_features, hidden_state, params):
    p = params
    src = edge_index[0]
    dst = edge_index[1]
    pad_n = EP - E
    idx_src = jnp.concatenate(
        [src, jnp.zeros((pad_n,), jnp.int32)]).reshape(EP // CH, CH)
    idx_dst = jnp.concatenate(
        [dst + N, jnp.full((pad_n,), N, jnp.int32)]).reshape(EP // CH, CH)
    dst_pad = jnp.concatenate(
        [dst, jnp.full((pad_n,), DUMP, jnp.int32)]).reshape(EP // CH, CH)
    zeros_acc = jnp.zeros((NP_, GH), jnp.float32)
    ones_vals = jnp.ones((CH, GH), jnp.float32)

    gather_f = _build_gather()
    segsum_f = _build_segsum(const_ones=False)
    degree_f = _build_segsum(const_ones=True)

    r1 = lambda v: v.reshape(1, -1)
    lyrs = p["layers"]
    splits = []
    for lyr in lyrs:
        splits.append((lyr["We1"][:NH], lyr["We1"][NH:2 * NH],
                       lyr["We1"][2 * NH:], lyr["Wn1"][:NH], lyr["Wn1"][NH:]))

    node_h, ptab = _tc_node_proj(node_features, p["Wnp"], r1(p["bnp"]),
                                 splits[0][0], splits[0][1])
    edge_h = _tc_edge_proj(optimizer_features, hidden_state,
                           p["Wep"][:EDGE_IN], p["Wep"][EDGE_IN:], r1(p["bep"]))

    degp = degree_f(ones_vals, dst_pad, zeros_acc)
    dinv = _tc_deginv(degp)

    n_rounds = L * R
    for t in range(n_rounds):
        lyr = lyrs[t // R]
        wa, wb, wc, wd, wf = splits[t // R]
        na, nb_ = splits[min(t + 1, n_rounds - 1) // R][:2]
        gath = gather_f(ptab, idx_src, idx_dst)
        edge_h = _tc_edge_mlp(gath, edge_h, wc, r1(lyr["be1"]),
                              lyr["We2"], r1(lyr["be2"]),
                              r1(lyr["ge"]), r1(lyr["be_ln"]))
        aggp = segsum_f(edge_h, dst_pad, zeros_acc)
        node_h, ptab = _tc_node_mlp(node_h, aggp, dinv, wd, wf, r1(lyr["bn1"]),
                                    lyr["Wn2"], r1(lyr["bn2"]),
                                    r1(lyr["gn"]), r1(lyr["bn_ln"]), na, nb_)

    updates, new_hidden = _tc_heads(
        edge_h, hidden_state, p["Wh1"], r1(p["bh1"]), p["Wh2"], r1(p["bh2"]),
        p["Wx"], p["Whh"], r1(p["bx"]), r1(p["bhh"]))
    return updates, new_hidden


# speedup vs baseline: 2.7647x; 1.0921x over previous
import functools

import jax
import jax.numpy as jnp
from jax import lax
from jax.experimental import pallas as pl
from jax.experimental.pallas import tpu as pltpu
from jax.experimental.pallas import tpu_sc as plsc

N = 10000
E = 320000
NODE_IN = 128
EDGE_IN = 16
NH = 64
EH = 64
GH = 128
L = 3
R = 2
HS = 32
SCALE = 1e-3

NC, NS = 2, 16
NW = NC * NS
CH = 128
RPC = 8
CHUNK = CH * RPC
HALF = CHUNK // 2
BE = 2000

NP_ = 10240
RZ = NP_ // NS
DUMP = N + 8
EP = ((E + CHUNK - 1) // CHUNK) * CHUNK
G2E = 2 * E


@functools.lru_cache(maxsize=None)
def _mesh():
    return plsc.VectorSubcoreMesh(
        core_axis_name="c", subcore_axis_name="s",
        num_cores=NC, num_subcores=NS)


def _wid():
    return lax.axis_index("s") * NC + lax.axis_index("c")


def _build_gather(interpret=False):
    n_chunks = EP // CHUNK
    k_max = (n_chunks + NW - 1) // NW
    NB = 4

    @functools.partial(
        pl.kernel, mesh=_mesh(), interpret=interpret,
        out_type=jax.ShapeDtypeStruct((EP, GH), jnp.float32),
        scratch_types=[
            pltpu.VMEM((RPC, CH), jnp.int32),
            pltpu.VMEM((RPC, CH), jnp.int32),
            [pltpu.VMEM((CH, GH), jnp.float32) for _ in range(NB)],
            [pltpu.SemaphoreType.DMA for _ in range(NB)],
            pltpu.SemaphoreType.DMA,
            [pltpu.SemaphoreType.DMA for _ in range(NB)],
        ],
    )
    def gather_k(table_hbm, idxs_hbm, idxd_hbm, out_hbm, idxs_v, idxd_v,
                 bufs, semb, sema, semo):
        wid = _wid()

        def drain_out(b):
            pltpu.make_async_copy(table_hbm.at[pl.ds(0, CH)], bufs[b],
                                  semo[b]).wait()

        def base_op(u):
            return pltpu.async_copy(table_hbm.at[idxs_v.at[u]],
                                    bufs[u % NB], semb[u % NB])

        def body(k, carry):
            chunk = wid + k * NW

            @pl.when(chunk < n_chunks)
            def _():
                rbase = pl.multiple_of(chunk * RPC, RPC)
                pltpu.sync_copy(idxs_hbm.at[pl.ds(rbase, RPC)], idxs_v)
                pltpu.sync_copy(idxd_hbm.at[pl.ds(rbase, RPC)], idxd_v)
                base_d = [None] * RPC
                for u in range(2):
                    pl.when(k > 0)(functools.partial(drain_out, u))
                    base_d[u] = base_op(u)
                for u in range(RPC):
                    base_d[u].wait()
                    add_d = pltpu.async_copy(
                        table_hbm.at[idxd_v.at[u]], bufs[u % NB], sema,
                        add=True)
                    v = u + 2
                    if v < RPC:
                        if v < NB:
                            pl.when(k > 0)(functools.partial(drain_out, v))
                        else:
                            drain_out(v % NB)
                        base_d[v] = base_op(v)
                    add_d.wait()
                    obase = pl.multiple_of(chunk * CHUNK + u * CH, CH)
                    pltpu.async_copy(bufs[u % NB],
                                     out_hbm.at[pl.ds(obase, CH)],
                                     semo[u % NB])

            return carry

        lax.fori_loop(0, k_max, body, 0)
        for b in range(NB):
            drain_out(b)

    return gather_k


def _build_segsum(const_ones, interpret=False):
    n_chunks = EP // CHUNK
    k_max = (n_chunks + NW - 1) // NW

    @functools.partial(
        pl.kernel, mesh=_mesh(), interpret=interpret,
        out_type=jax.ShapeDtypeStruct((NC * NP_, GH), jnp.float32),
        scratch_types=[
            pltpu.VMEM((RPC, CH), jnp.int32),
            [pltpu.VMEM((CH, GH), jnp.float32) for _ in range(2)],
            [pltpu.SemaphoreType.DMA for _ in range(2)],
            [pltpu.SemaphoreType.DMA for _ in range(2)],
            pltpu.VMEM_SHARED((NP_, GH), jnp.float32),
        ],
    )
    def segsum_k(vals_hbm, idx_hbm, zeros_hbm, out_hbm, idx_v, bufs, semv,
                 sems, acc_sh):
        cid = lax.axis_index("c")
        sid = lax.axis_index("s")
        wid = _wid()
        zbase = pl.multiple_of(sid * RZ, RZ)
        pltpu.sync_copy(zeros_hbm.at[pl.ds(zbase, RZ)],
                        acc_sh.at[pl.ds(zbase, RZ)])
        if const_ones:
            pltpu.sync_copy(vals_hbm, bufs[0])
        plsc.subcore_barrier()

        def drain_add(b):
            pltpu.make_async_copy(vals_hbm.at[pl.ds(0, CH)], bufs[b],
                                  sems[b]).wait()

        def vals_op(chunk, u):
            ebase = pl.multiple_of(chunk * CHUNK + u * CH, CH)
            return pltpu.async_copy(vals_hbm.at[pl.ds(ebase, CH)],
                                    bufs[u % 2], semv[u % 2])

        def body(k, carry):
            chunk = wid + k * NW

            @pl.when(chunk < n_chunks)
            def _():
                rbase = pl.multiple_of(chunk * RPC, RPC)
                pltpu.sync_copy(idx_hbm.at[pl.ds(rbase, RPC)], idx_v)
                if const_ones:
                    descs = [pltpu.async_copy(bufs[0],
                                              acc_sh.at[idx_v.at[u]],
                                              sems[0], add=True)
                             for u in range(RPC)]
                    for d in descs:
                        d.wait()
                else:
                    pl.when(k > 0)(functools.partial(drain_add, 0))
                    vals_d = [None] * RPC
                    vals_d[0] = vals_op(chunk, 0)
                    for u in range(RPC):
                        v = u + 1
                        if v < RPC:
                            if v == 1:
                                pl.when(k > 0)(
                                    functools.partial(drain_add, 1))
                            else:
                                drain_add(v % 2)
                            vals_d[v] = vals_op(chunk, v)
                        vals_d[u].wait()
                        pltpu.async_copy(bufs[u % 2],
                                         acc_sh.at[idx_v.at[u]],
                                         sems[u % 2], add=True)

            return carry

        lax.fori_loop(0, k_max, body, 0)
        if not const_ones:
            drain_add(0)
            drain_add(1)
        plsc.subcore_barrier()
        obase = pl.multiple_of(cid * NP_ + sid * RZ, RZ)
        pltpu.sync_copy(acc_sh.at[pl.ds(zbase, RZ)],
                        out_hbm.at[pl.ds(obase, RZ)])

    return segsum_k


def _gelu(x):
    return jax.nn.gelu(x)


def _dot(a, b):
    return jnp.dot(a, b, preferred_element_type=jnp.float32)


def _ln_apply(x, g, b):
    mu = jnp.mean(x, axis=-1, keepdims=True)
    var = jnp.mean((x - mu) ** 2, axis=-1, keepdims=True)
    return (x - mu) / jnp.sqrt(var + 1e-5) * g + b


def _proj_table(nh, wa, wb):
    return jnp.concatenate([_dot(nh, wa), _dot(nh, wb)], axis=0)


def _tc_node_proj(nf, w, b, wa, wb):
    def body(nf_r, w_r, b_r, wa_r, wb_r, o_r, p_r):
        nh = _gelu(_dot(nf_r[...], w_r[...]) + b_r[...])
        o_r[...] = nh
        p_r[...] = _proj_table(nh, wa_r[...], wb_r[...])
    return pl.pallas_call(
        body,
        out_shape=[jax.ShapeDtypeStruct((N, NH), jnp.float32),
                   jax.ShapeDtypeStruct((2 * N, GH), jnp.float32)],
    )(nf, w, b, wa, wb)


def _tc_edge_proj(opt, hid, wa, wb, b):
    def body(o_r, h_r, wa_r, wb_r, b_r, out_r):
        x = _gelu(_dot(o_r[...], wa_r[...]) +
                  _dot(h_r[...], wb_r[...]) + b_r[...])
        out_r[...] = jnp.concatenate([x, x], axis=-1)
    nb = E // BE
    return pl.pallas_call(
        body,
        grid=(nb,),
        in_specs=[
            pl.BlockSpec((BE, EDGE_IN), lambda i: (i, 0)),
            pl.BlockSpec((BE, HS), lambda i: (i, 0)),
            pl.BlockSpec((EDGE_IN, EH), lambda i: (0, 0)),
            pl.BlockSpec((HS, EH), lambda i: (0, 0)),
            pl.BlockSpec((1, EH), lambda i: (0, 0)),
        ],
        out_specs=pl.BlockSpec((BE, GH), lambda i: (i, 0)),
        out_shape=jax.ShapeDtypeStruct((EP, GH), jnp.float32),
    )(opt, hid, wa, wb, b)


def _tc_edge_mlp(gath, eh, wc, b1, w2, b2, g, bl):
    nb = E // BE

    def body(g_sum_r, e_r, wc_r, b1_r, w2_r, b2_r, g_r, bl_r, out_r):
        e_v = e_r[...][:, :EH]
        h = g_sum_r[...] + _dot(e_v, wc_r[...]) + b1_r[...]
        m = _dot(_gelu(h), w2_r[...]) + b2_r[...]
        x = _ln_apply(e_v + m, g_r[...], bl_r[...])
        out_r[...] = jnp.concatenate([x, jnp.ones_like(x)], axis=-1)

    return pl.pallas_call(
        body,
        grid=(nb,),
        in_specs=[
            pl.BlockSpec((BE, GH), lambda i: (i, 0)),
            pl.BlockSpec((BE, GH), lambda i: (i, 0)),
            pl.BlockSpec((EH, GH), lambda i: (0, 0)),
            pl.BlockSpec((1, GH), lambda i: (0, 0)),
            pl.BlockSpec((GH, EH), lambda i: (0, 0)),
            pl.BlockSpec((1, EH), lambda i: (0, 0)),
            pl.BlockSpec((1, EH), lambda i: (0, 0)),
            pl.BlockSpec((1, EH), lambda i: (0, 0)),
        ],
        out_specs=pl.BlockSpec((BE, GH), lambda i: (i, 0)),
        out_shape=jax.ShapeDtypeStruct((EP, GH), jnp.float32),
    )(gath, eh, wc, b1, w2, b2, g, bl)


def _tc_deginv(degp):
    def body(d_r, o_r):
        d = d_r[...]
        o_r[...] = 1.0 / jnp.maximum(d[:N, NH:] + d[NP_:NP_ + N, NH:], 1.0)
    return pl.pallas_call(
        body, out_shape=jax.ShapeDtypeStruct((N, NH), jnp.float32),
    )(degp)


def _tc_node_mlp(nh, aggp, dinv, wd, wf, b1, w2, b2, g, bl, wa, wb):
    def body(nh_r, a_r, di_r, wd_r, wf_r, b1_r, w2_r, b2_r, g_r, bl_r,
             wa_r, wb_r, out_r, p_r):
        nh_v = nh_r[...]
        a = a_r[...]
        agg = (a[:N, :NH] + a[NP_:NP_ + N, :NH]) * di_r[...]
        h = _gelu(_dot(nh_v, wd_r[...]) + _dot(agg, wf_r[...]) + b1_r[...])
        u = _dot(h, w2_r[...]) + b2_r[...]
        nh_new = _ln_apply(nh_v + u, g_r[...], bl_r[...])
        out_r[...] = nh_new
        p_r[...] = _proj_table(nh_new, wa_r[...], wb_r[...])
    return pl.pallas_call(
        body,
        out_shape=[jax.ShapeDtypeStruct((N, NH), jnp.float32),
                   jax.ShapeDtypeStruct((2 * N, GH), jnp.float32)],
    )(nh, aggp, dinv, wd, wf, b1, w2, b2, g, bl, wa, wb)


def _tc_heads(eh, hid, wh1, bh1, wh2, bh2, wx, whh, bx, bhh):
    nb = E // BE

    def body(e_r, h_r, wh1_r, bh1_r, wh2_r, bh2_r, wx_r, whh_r, bx_r, bhh_r,
             u_r, nh_r):
        e_v = e_r[...][:, :EH]
        h_v = h_r[...]
        t = _gelu(_dot(e_v, wh1_r[...]) + bh1_r[...])
        u_r[...] = (_dot(t, wh2_r[...]) + bh2_r[...]) * SCALE
        gx = _dot(e_v, wx_r[...]) + bx_r[...]
        gh = _dot(h_v, whh_r[...]) + bhh_r[...]
        r = jax.nn.sigmoid(gx[:, :HS] + gh[:, :HS])
        z = jax.nn.sigmoid(gx[:, HS:2 * HS] + gh[:, HS:2 * HS])
        n = jnp.tanh(gx[:, 2 * HS:] + r * gh[:, 2 * HS:])
        nh_r[...] = (1.0 - z) * n + z * h_v

    return pl.pallas_call(
        body,
        grid=(nb,),
        in_specs=[
            pl.BlockSpec((BE, GH), lambda i: (i, 0)),
            pl.BlockSpec((BE, HS), lambda i: (i, 0)),
            pl.BlockSpec((EH, GH), lambda i: (0, 0)),
            pl.BlockSpec((1, GH), lambda i: (0, 0)),
            pl.BlockSpec((GH, 1), lambda i: (0, 0)),
            pl.BlockSpec((1, 1), lambda i: (0, 0)),
            pl.BlockSpec((EH, 3 * HS), lambda i: (0, 0)),
            pl.BlockSpec((HS, 3 * HS), lambda i: (0, 0)),
            pl.BlockSpec((1, 3 * HS), lambda i: (0, 0)),
            pl.BlockSpec((1, 3 * HS), lambda i: (0, 0)),
        ],
        out_specs=[
            pl.BlockSpec((BE, 1), lambda i: (i, 0)),
            pl.BlockSpec((BE, HS), lambda i: (i, 0)),
        ],
        out_shape=[
            jax.ShapeDtypeStruct((E, 1), jnp.float32),
            jax.ShapeDtypeStruct((E, HS), jnp.float32),
        ],
    )(eh, hid, wh1, bh1, wh2, bh2, wx, whh, bx, bhh)


def kernel(node_features, edge_index, optimizer_features, hidden_state, params):
    p = params
    src = edge_index[0]
    dst = edge_index[1]
    pad_n = EP - E
    idx_src = jnp.concatenate(
        [src, jnp.zeros((pad_n,), jnp.int32)]).reshape(EP // CH, CH)
    idx_dst = jnp.concatenate(
        [dst + N, jnp.full((pad_n,), N, jnp.int32)]).reshape(EP // CH, CH)
    dst_pad = jnp.concatenate(
        [dst, jnp.full((pad_n,), DUMP, jnp.int32)]).reshape(EP // CH, CH)
    zeros_acc = jnp.zeros((NP_, GH), jnp.float32)

    gather_f = _build_gather()
    segsum_f = _build_segsum(const_ones=False)

    r1 = lambda v: v.reshape(1, -1)
    lyrs = p["layers"]
    splits = []
    for lyr in lyrs:
        splits.append((lyr["We1"][:NH], lyr["We1"][NH:2 * NH],
                       lyr["We1"][2 * NH:], lyr["Wn1"][:NH], lyr["Wn1"][NH:]))

    node_h, ptab = _tc_node_proj(node_features, p["Wnp"], r1(p["bnp"]),
                                 splits[0][0], splits[0][1])
    edge_h = _tc_edge_proj(optimizer_features, hidden_state,
                           p["Wep"][:EDGE_IN], p["Wep"][EDGE_IN:], r1(p["bep"]))

    n_rounds = L * R
    dinv = None
    for t in range(n_rounds):
        lyr = lyrs[t // R]
        wa, wb, wc, wd, wf = splits[t // R]
        na, nb_ = splits[min(t + 1, n_rounds - 1) // R][:2]
        gath = gather_f(ptab, idx_src, idx_dst)
        edge_h = _tc_edge_mlp(gath, edge_h, wc, r1(lyr["be1"]),
                              lyr["We2"], r1(lyr["be2"]),
                              r1(lyr["ge"]), r1(lyr["be_ln"]))
        aggp = segsum_f(edge_h, dst_pad, zeros_acc)
        if dinv is None:
            dinv = _tc_deginv(aggp)
        node_h, ptab = _tc_node_mlp(node_h, aggp, dinv, wd, wf, r1(lyr["bn1"]),
                                    lyr["Wn2"], r1(lyr["bn2"]),
                                    r1(lyr["gn"]), r1(lyr["bn_ln"]), na, nb_)

    updates, new_hidden = _tc_heads(
        edge_h, hidden_state, p["Wh1"], r1(p["bh1"]), p["Wh2"], r1(p["bh2"]),
        p["Wx"], p["Whh"], r1(p["bx"]), r1(p["bhh"]))
    return updates, new_hidden


# speedup vs baseline: 2.9139x; 1.0540x over previous
import functools

import jax
import jax.numpy as jnp
from jax import lax
from jax.experimental import pallas as pl
from jax.experimental.pallas import tpu as pltpu
from jax.experimental.pallas import tpu_sc as plsc

N = 10000
E = 320000
NODE_IN = 128
EDGE_IN = 16
NH = 64
EH = 64
GH = 128
L = 3
R = 2
HS = 32
SCALE = 1e-3

NC, NS = 2, 16
NW = NC * NS
CH = 128
RPC = 8
CHUNK = CH * RPC
HALF = CHUNK // 2
BE = 2000

NP_ = 10240
RZ = NP_ // NS
DUMP = N + 8
EP = ((E + CHUNK - 1) // CHUNK) * CHUNK
G2E = 2 * E


@functools.lru_cache(maxsize=None)
def _mesh():
    return plsc.VectorSubcoreMesh(
        core_axis_name="c", subcore_axis_name="s",
        num_cores=NC, num_subcores=NS)


def _wid():
    return lax.axis_index("s") * NC + lax.axis_index("c")


def _build_gather(interpret=False):
    n_chunks = EP // CHUNK
    k_max = (n_chunks + NW - 1) // NW
    NB = 4

    @functools.partial(
        pl.kernel, mesh=_mesh(), interpret=interpret,
        out_type=jax.ShapeDtypeStruct((EP, GH), jnp.float32),
        scratch_types=[
            pltpu.VMEM((RPC, CH), jnp.int32),
            pltpu.VMEM((RPC, CH), jnp.int32),
            [pltpu.VMEM((CH, GH), jnp.float32) for _ in range(NB)],
            [pltpu.SemaphoreType.DMA for _ in range(NB)],
            pltpu.SemaphoreType.DMA,
            [pltpu.SemaphoreType.DMA for _ in range(NB)],
        ],
    )
    def gather_k(table_hbm, idxs_hbm, idxd_hbm, out_hbm, idxs_v, idxd_v,
                 bufs, semb, sema, semo):
        wid = _wid()

        def drain_out(b):
            pltpu.make_async_copy(table_hbm.at[pl.ds(0, CH)], bufs[b],
                                  semo[b]).wait()

        def base_op(u):
            return pltpu.async_copy(table_hbm.at[idxs_v.at[u]],
                                    bufs[u % NB], semb[u % NB])

        def body(k, carry):
            chunk = wid + k * NW

            @pl.when(chunk < n_chunks)
            def _():
                rbase = pl.multiple_of(chunk * RPC, RPC)
                pltpu.sync_copy(idxs_hbm.at[pl.ds(rbase, RPC)], idxs_v)
                pltpu.sync_copy(idxd_hbm.at[pl.ds(rbase, RPC)], idxd_v)
                base_d = [None] * RPC
                for u in range(2):
                    pl.when(k > 0)(functools.partial(drain_out, u))
                    base_d[u] = base_op(u)
                for u in range(RPC):
                    base_d[u].wait()
                    add_d = pltpu.async_copy(
                        table_hbm.at[idxd_v.at[u]], bufs[u % NB], sema,
                        add=True)
                    v = u + 2
                    if v < RPC:
                        if v < NB:
                            pl.when(k > 0)(functools.partial(drain_out, v))
                        else:
                            drain_out(v % NB)
                        base_d[v] = base_op(v)
                    add_d.wait()
                    obase = pl.multiple_of(chunk * CHUNK + u * CH, CH)
                    pltpu.async_copy(bufs[u % NB],
                                     out_hbm.at[pl.ds(obase, CH)],
                                     semo[u % NB])

            return carry

        lax.fori_loop(0, k_max, body, 0)
        for b in range(NB):
            drain_out(b)

    return gather_k


def _build_segsum(const_ones, interpret=False):
    n_chunks = EP // CHUNK
    k_max = (n_chunks + NW - 1) // NW

    @functools.partial(
        pl.kernel, mesh=_mesh(), interpret=interpret,
        out_type=jax.ShapeDtypeStruct((NC * NP_, GH), jnp.float32),
        scratch_types=[
            pltpu.VMEM((RPC, CH), jnp.int32),
            [pltpu.VMEM((CH, GH), jnp.float32) for _ in range(2)],
            [pltpu.SemaphoreType.DMA for _ in range(2)],
            [pltpu.SemaphoreType.DMA for _ in range(2)],
            pltpu.VMEM_SHARED((NP_, GH), jnp.float32),
        ],
    )
    def segsum_k(vals_hbm, idx_hbm, zeros_hbm, out_hbm, idx_v, bufs, semv,
                 sems, acc_sh):
        cid = lax.axis_index("c")
        sid = lax.axis_index("s")
        wid = _wid()
        zbase = pl.multiple_of(sid * RZ, RZ)
        pltpu.sync_copy(zeros_hbm.at[pl.ds(zbase, RZ)],
                        acc_sh.at[pl.ds(zbase, RZ)])
        if const_ones:
            pltpu.sync_copy(vals_hbm, bufs[0])
        plsc.subcore_barrier()

        def drain_add(b):
            pltpu.make_async_copy(vals_hbm.at[pl.ds(0, CH)], bufs[b],
                                  sems[b]).wait()

        def vals_op(chunk, u):
            ebase = pl.multiple_of(chunk * CHUNK + u * CH, CH)
            return pltpu.async_copy(vals_hbm.at[pl.ds(ebase, CH)],
                                    bufs[u % 2], semv[u % 2])

        def body(k, carry):
            chunk = wid + k * NW

            @pl.when(chunk < n_chunks)
            def _():
                rbase = pl.multiple_of(chunk * RPC, RPC)
                pltpu.sync_copy(idx_hbm.at[pl.ds(rbase, RPC)], idx_v)
                if const_ones:
                    descs = [pltpu.async_copy(bufs[0],
                                              acc_sh.at[idx_v.at[u]],
                                              sems[0], add=True)
                             for u in range(RPC)]
                    for d in descs:
                        d.wait()
                else:
                    pl.when(k > 0)(functools.partial(drain_add, 0))
                    vals_d = [None] * RPC
                    vals_d[0] = vals_op(chunk, 0)
                    for u in range(RPC):
                        v = u + 1
                        if v < RPC:
                            if v == 1:
                                pl.when(k > 0)(
                                    functools.partial(drain_add, 1))
                            else:
                                drain_add(v % 2)
                            vals_d[v] = vals_op(chunk, v)
                        vals_d[u].wait()
                        pltpu.async_copy(bufs[u % 2],
                                         acc_sh.at[idx_v.at[u]],
                                         sems[u % 2], add=True)

            return carry

        lax.fori_loop(0, k_max, body, 0)
        if not const_ones:
            drain_add(0)
            drain_add(1)
        plsc.subcore_barrier()
        obase = pl.multiple_of(cid * NP_ + sid * RZ, RZ)
        pltpu.sync_copy(acc_sh.at[pl.ds(zbase, RZ)],
                        out_hbm.at[pl.ds(obase, RZ)])

    return segsum_k


def _gelu(x):
    return jax.nn.gelu(x)


def _dot(a, b):
    return jnp.dot(a, b, preferred_element_type=jnp.float32)


def _ln_apply(x, g, b):
    mu = jnp.mean(x, axis=-1, keepdims=True)
    var = jnp.mean((x - mu) ** 2, axis=-1, keepdims=True)
    return (x - mu) / jnp.sqrt(var + 1e-5) * g + b


def _proj_table(nh, wa, wb):
    return jnp.concatenate([_dot(nh, wa), _dot(nh, wb)], axis=0)


def _tc_node_proj(nf, w, b, wa, wb):
    def body(nf_r, w_r, b_r, wa_r, wb_r, o_r, p_r):
        nh = _gelu(_dot(nf_r[...], w_r[...]) + b_r[...])
        o_r[...] = nh
        p_r[...] = _proj_table(nh, wa_r[...], wb_r[...])
    return pl.pallas_call(
        body,
        out_shape=[jax.ShapeDtypeStruct((N, NH), jnp.float32),
                   jax.ShapeDtypeStruct((2 * N, GH), jnp.float32)],
    )(nf, w, b, wa, wb)


def _edge_mlp_math(g_sum, e_v, wc, b1, w2, b2, g, bl):
    h = g_sum + _dot(e_v, wc) + b1
    m = _dot(_gelu(h), w2) + b2
    return _ln_apply(e_v + m, g, bl)


_W_SPECS = [
    pl.BlockSpec((EH, GH), lambda i: (0, 0)),
    pl.BlockSpec((1, GH), lambda i: (0, 0)),
    pl.BlockSpec((GH, EH), lambda i: (0, 0)),
    pl.BlockSpec((1, EH), lambda i: (0, 0)),
    pl.BlockSpec((1, EH), lambda i: (0, 0)),
    pl.BlockSpec((1, EH), lambda i: (0, 0)),
]


def _tc_edge_first(gath, opt, hid, wpa, wpb, bp, wc, b1, w2, b2, g, bl):
    nb = E // BE

    def body(g_sum_r, o_r, h_r, wpa_r, wpb_r, bp_r, wc_r, b1_r, w2_r, b2_r,
             g_r, bl_r, out_r):
        e_v = _gelu(_dot(o_r[...], wpa_r[...]) +
                    _dot(h_r[...], wpb_r[...]) + bp_r[...])
        x = _edge_mlp_math(g_sum_r[...], e_v, wc_r[...], b1_r[...], w2_r[...],
                           b2_r[...], g_r[...], bl_r[...])
        out_r[...] = jnp.concatenate([x, jnp.ones_like(x)], axis=-1)

    return pl.pallas_call(
        body,
        grid=(nb,),
        in_specs=[
            pl.BlockSpec((BE, GH), lambda i: (i, 0)),
            pl.BlockSpec((BE, EDGE_IN), lambda i: (i, 0)),
            pl.BlockSpec((BE, HS), lambda i: (i, 0)),
            pl.BlockSpec((EDGE_IN, EH), lambda i: (0, 0)),
            pl.BlockSpec((HS, EH), lambda i: (0, 0)),
            pl.BlockSpec((1, EH), lambda i: (0, 0)),
        ] + _W_SPECS,
        out_specs=pl.BlockSpec((BE, GH), lambda i: (i, 0)),
        out_shape=jax.ShapeDtypeStruct((EP, GH), jnp.float32),
    )(gath, opt, hid, wpa, wpb, bp, wc, b1, w2, b2, g, bl)


def _tc_edge_mlp(gath, eh, wc, b1, w2, b2, g, bl):
    nb = E // BE

    def body(g_sum_r, e_r, wc_r, b1_r, w2_r, b2_r, g_r, bl_r, out_r):
        e_v = e_r[...][:, :EH]
        x = _edge_mlp_math(g_sum_r[...], e_v, wc_r[...], b1_r[...], w2_r[...],
                           b2_r[...], g_r[...], bl_r[...])
        out_r[...] = jnp.concatenate([x, jnp.ones_like(x)], axis=-1)

    return pl.pallas_call(
        body,
        grid=(nb,),
        in_specs=[
            pl.BlockSpec((BE, GH), lambda i: (i, 0)),
            pl.BlockSpec((BE, GH), lambda i: (i, 0)),
        ] + _W_SPECS,
        out_specs=pl.BlockSpec((BE, GH), lambda i: (i, 0)),
        out_shape=jax.ShapeDtypeStruct((EP, GH), jnp.float32),
    )(gath, eh, wc, b1, w2, b2, g, bl)


def _tc_edge_last(gath, eh, wc, b1, w2, b2, g, bl, hid,
                  wh1, bh1, wh2, bh2, wx, whh, bx, bhh):
    nb = E // BE

    def body(g_sum_r, e_r, wc_r, b1_r, w2_r, b2_r, g_r, bl_r, h_r,
             wh1_r, bh1_r, wh2_r, bh2_r, wx_r, whh_r, bx_r, bhh_r,
             u_r, nh_r):
        e_v = e_r[...][:, :EH]
        x = _edge_mlp_math(g_sum_r[...], e_v, wc_r[...], b1_r[...], w2_r[...],
                           b2_r[...], g_r[...], bl_r[...])
        h_v = h_r[...]
        t = _gelu(_dot(x, wh1_r[...]) + bh1_r[...])
        u_r[...] = (_dot(t, wh2_r[...]) + bh2_r[...]) * SCALE
        gx = _dot(x, wx_r[...]) + bx_r[...]
        gh = _dot(h_v, whh_r[...]) + bhh_r[...]
        r = jax.nn.sigmoid(gx[:, :HS] + gh[:, :HS])
        z = jax.nn.sigmoid(gx[:, HS:2 * HS] + gh[:, HS:2 * HS])
        n = jnp.tanh(gx[:, 2 * HS:] + r * gh[:, 2 * HS:])
        nh_r[...] = (1.0 - z) * n + z * h_v

    return pl.pallas_call(
        body,
        grid=(nb,),
        in_specs=[
            pl.BlockSpec((BE, GH), lambda i: (i, 0)),
            pl.BlockSpec((BE, GH), lambda i: (i, 0)),
        ] + _W_SPECS + [
            pl.BlockSpec((BE, HS), lambda i: (i, 0)),
            pl.BlockSpec((EH, GH), lambda i: (0, 0)),
            pl.BlockSpec((1, GH), lambda i: (0, 0)),
            pl.BlockSpec((GH, 1), lambda i: (0, 0)),
            pl.BlockSpec((1, 1), lambda i: (0, 0)),
            pl.BlockSpec((EH, 3 * HS), lambda i: (0, 0)),
            pl.BlockSpec((HS, 3 * HS), lambda i: (0, 0)),
            pl.BlockSpec((1, 3 * HS), lambda i: (0, 0)),
            pl.BlockSpec((1, 3 * HS), lambda i: (0, 0)),
        ],
        out_specs=[
            pl.BlockSpec((BE, 1), lambda i: (i, 0)),
            pl.BlockSpec((BE, HS), lambda i: (i, 0)),
        ],
        out_shape=[
            jax.ShapeDtypeStruct((E, 1), jnp.float32),
            jax.ShapeDtypeStruct((E, HS), jnp.float32),
        ],
    )(gath, eh, wc, b1, w2, b2, g, bl, hid,
      wh1, bh1, wh2, bh2, wx, whh, bx, bhh)


def _tc_deginv(degp):
    def body(d_r, o_r):
        d = d_r[...]
        o_r[...] = 1.0 / jnp.maximum(d[:N, NH:] + d[NP_:NP_ + N, NH:], 1.0)
    return pl.pallas_call(
        body, out_shape=jax.ShapeDtypeStruct((N, NH), jnp.float32),
    )(degp)


def _tc_node_mlp(nh, aggp, dinv, wd, wf, b1, w2, b2, g, bl, wa, wb):
    def body(nh_r, a_r, di_r, wd_r, wf_r, b1_r, w2_r, b2_r, g_r, bl_r,
             wa_r, wb_r, out_r, p_r):
        nh_v = nh_r[...]
        a = a_r[...]
        agg = (a[:N, :NH] + a[NP_:NP_ + N, :NH]) * di_r[...]
        h = _gelu(_dot(nh_v, wd_r[...]) + _dot(agg, wf_r[...]) + b1_r[...])
        u = _dot(h, w2_r[...]) + b2_r[...]
        nh_new = _ln_apply(nh_v + u, g_r[...], bl_r[...])
        out_r[...] = nh_new
        p_r[...] = _proj_table(nh_new, wa_r[...], wb_r[...])
    return pl.pallas_call(
        body,
        out_shape=[jax.ShapeDtypeStruct((N, NH), jnp.float32),
                   jax.ShapeDtypeStruct((2 * N, GH), jnp.float32)],
    )(nh, aggp, dinv, wd, wf, b1, w2, b2, g, bl, wa, wb)


def kernel(node_features, edge_index, optimizer_features, hidden_state, params):
    p = params
    src = edge_index[0]
    dst = edge_index[1]
    pad_n = EP - E
    idx_src = jnp.concatenate(
        [src, jnp.zeros((pad_n,), jnp.int32)]).reshape(EP // CH, CH)
    idx_dst = jnp.concatenate(
        [dst + N, jnp.full((pad_n,), N, jnp.int32)]).reshape(EP // CH, CH)
    dst_pad = jnp.concatenate(
        [dst, jnp.full((pad_n,), DUMP, jnp.int32)]).reshape(EP // CH, CH)
    zeros_acc = jnp.zeros((NP_, GH), jnp.float32)

    gather_f = _build_gather()
    segsum_f = _build_segsum(const_ones=False)

    r1 = lambda v: v.reshape(1, -1)
    lyrs = p["layers"]
    splits = []
    for lyr in lyrs:
        splits.append((lyr["We1"][:NH], lyr["We1"][NH:2 * NH],
                       lyr["We1"][2 * NH:], lyr["Wn1"][:NH], lyr["Wn1"][NH:]))

    node_h, ptab = _tc_node_proj(node_features, p["Wnp"], r1(p["bnp"]),
                                 splits[0][0], splits[0][1])

    n_rounds = L * R
    dinv = None
    edge_h = None
    for t in range(n_rounds):
        lyr = lyrs[t // R]
        wa, wb, wc, wd, wf = splits[t // R]
        na, nb_ = splits[min(t + 1, n_rounds - 1) // R][:2]
        gath = gather_f(ptab, idx_src, idx_dst)
        mlp_w = (wc, r1(lyr["be1"]), lyr["We2"], r1(lyr["be2"]),
                 r1(lyr["ge"]), r1(lyr["be_ln"]))
        if t == 0:
            edge_h = _tc_edge_first(gath, optimizer_features, hidden_state,
                                    p["Wep"][:EDGE_IN], p["Wep"][EDGE_IN:],
                                    r1(p["bep"]), *mlp_w)
        elif t < n_rounds - 1:
            edge_h = _tc_edge_mlp(gath, edge_h, *mlp_w)
        else:
            return _tc_edge_last(
                gath, edge_h, *mlp_w, hidden_state,
                p["Wh1"], r1(p["bh1"]), p["Wh2"], r1(p["bh2"]),
                p["Wx"], p["Whh"], r1(p["bx"]), r1(p["bhh"]))
        aggp = segsum_f(edge_h, dst_pad, zeros_acc)
        if dinv is None:
            dinv = _tc_deginv(aggp)
        node_h, ptab = _tc_node_mlp(node_h, aggp, dinv, wd, wf, r1(lyr["bn1"]),
                                    lyr["Wn2"], r1(lyr["bn2"]),
                                    r1(lyr["gn"]), r1(lyr["bn_ln"]), na, nb_)
